# baseline (speedup 1.0000x reference)
import dataclasses
import functools

import jax
import jax.numpy as jnp
from jax import lax
from jax.experimental import pallas as pl
from jax.experimental.pallas import tpu as pltpu
from jax.experimental.pallas import tpu_sc as plsc

N_NODES = 10000
N_EDGES = 320000
D_FEAT = 128

NC = 2
NS = 16
NW = NC * NS
CHUNK = 128
TOTAL_CHUNKS = N_EDGES // CHUNK
MAX_CHUNKS = -(-TOTAL_CHUNKS // NW)
LANES = 16
DEPTH = 3
WO_ROWS = 80


def _sc_body(h_hbm, ei_hbm, w_hbm, out_hbm, idx_v, w_v, rows_v, acc_sh,
             *sems):
    isem = sems[0:DEPTH]
    gsem = sems[DEPTH:2 * DEPTH]
    ssem = sems[2 * DEPTH:3 * DEPTH]

    c = lax.axis_index("c")
    s = lax.axis_index("s")
    wid = c * NS + s
    nchunks = jnp.where(wid < TOTAL_CHUNKS - (MAX_CHUNKS - 1) * NW,
                        MAX_CHUNKS, MAX_CHUNKS - 1)

    def issue_idx(n, j):
        off = (wid + n * NW) * CHUNK
        pltpu.async_copy(ei_hbm.at[:, pl.ds(off, CHUNK)],
                         idx_v.at[j], isem[j])
        pltpu.async_copy(w_hbm.at[0, pl.ds(off, CHUNK)], w_v.at[j], isem[j])

    def wait_idx(j):
        pltpu.make_async_copy(ei_hbm.at[:, pl.ds(0, CHUNK)], idx_v.at[j],
                              isem[j]).wait()
        pltpu.make_async_copy(w_hbm.at[0, pl.ds(0, CHUNK)], w_v.at[j],
                              isem[j]).wait()

    def issue_gather(j):
        pltpu.async_copy(h_hbm.at[idx_v.at[j, 0]], rows_v.at[j], gsem[j])

    def wait_gather(j):
        pltpu.make_async_copy(h_hbm.at[idx_v.at[j, 0]], rows_v.at[j],
                              gsem[j]).wait()

    def issue_scatter(j):
        pltpu.async_copy(rows_v.at[j], acc_sh.at[idx_v.at[j, 1]], ssem[j],
                         add=True)

    def wait_scatter(j):
        pltpu.make_async_copy(rows_v.at[j], acc_sh.at[idx_v.at[j, 1]],
                              ssem[j]).wait()

    issue_idx(0, 0)
    issue_idx(1, 1)
    wait_idx(0)
    issue_gather(0)

    row_base = s * 640
    wo_trips = jnp.where(s == NS - 1, 5, 8)

    @pl.loop(0, WO_ROWS)
    def _zero_stage(i):
        for k in range(D_FEAT // LANES):
            rows_v[2, i, pl.ds(k * LANES, LANES)] = jnp.zeros((LANES,),
                                                              jnp.float32)

    def _zero_acc(j, _):
        pltpu.async_copy(rows_v.at[2, pl.ds(0, WO_ROWS)],
                         acc_sh.at[pl.ds(row_base + j * WO_ROWS, WO_ROWS)],
                         ssem[0])
        return _

    lax.fori_loop(0, wo_trips, _zero_acc, None)

    def _zero_drain(j, _):
        pltpu.make_async_copy(
            rows_v.at[2, pl.ds(0, WO_ROWS)],
            acc_sh.at[pl.ds(row_base, WO_ROWS)], ssem[0]).wait()
        return _

    lax.fori_loop(0, wo_trips, _zero_drain, None)

    plsc.subcore_barrier()

    @pl.loop(0, (MAX_CHUNKS + DEPTH - 1) // DEPTH)
    def _triple(m):
        for b in range(DEPTH):
            n = m * DEPTH + b

            @pl.when(n < nchunks)
            def _():
                @pl.when(n + 1 < nchunks)
                def _():
                    wait_idx((b + 1) % DEPTH)
                    issue_gather((b + 1) % DEPTH)

                wait_gather(b)

                @pl.when(n >= 1)
                def _():
                    wait_scatter((b + 2) % DEPTH)

                @pl.when(n + 2 < nchunks)
                def _():
                    issue_idx(n + 2, (b + 2) % DEPTH)

                @pl.loop(0, CHUNK // LANES)
                def _scale(g):
                    wv = w_v[b, pl.ds(g * LANES, LANES)]
                    for jj in range(LANES):
                        w = wv[jj]
                        i = g * LANES + jj
                        for k in range(D_FEAT // LANES):
                            sl = pl.ds(k * LANES, LANES)
                            rows_v[b, i, sl] = rows_v[b, i, sl] * w

                issue_scatter(b)

    @pl.when(nchunks == MAX_CHUNKS)
    def _():
        wait_scatter((MAX_CHUNKS - 1) % DEPTH)

    @pl.when(nchunks == MAX_CHUNKS - 1)
    def _():
        wait_scatter((MAX_CHUNKS - 2) % DEPTH)

    plsc.subcore_barrier()

    def _writeout(j, _):
        row0 = row_base + j * WO_ROWS
        pltpu.async_copy(acc_sh.at[pl.ds(row0, WO_ROWS)],
                         out_hbm.at[c, pl.ds(row0, WO_ROWS)], ssem[0])
        return _

    lax.fori_loop(0, wo_trips, _writeout, None)

    def _writeout_drain(j, _):
        pltpu.make_async_copy(
            acc_sh.at[pl.ds(row_base, WO_ROWS)],
            out_hbm.at[c, pl.ds(row_base, WO_ROWS)], ssem[0]).wait()
        return _

    lax.fori_loop(0, wo_trips, _writeout_drain, None)


_cp = pltpu.CompilerParams()
if "needs_layout_passes" in pltpu.CompilerParams.__dataclass_fields__:
    _cp = dataclasses.replace(_cp, needs_layout_passes=False)

_sc_call = functools.partial(
    pl.kernel,
    compiler_params=_cp,
    out_type=jax.ShapeDtypeStruct((NC, N_NODES, D_FEAT), jnp.float32),
    mesh=plsc.VectorSubcoreMesh(core_axis_name="c", subcore_axis_name="s"),
    scratch_types=[
        pltpu.VMEM((DEPTH, 2, CHUNK), jnp.int32),
        pltpu.VMEM((DEPTH, CHUNK), jnp.float32),
        pltpu.VMEM((DEPTH, CHUNK, D_FEAT), jnp.float32),
        pltpu.VMEM_SHARED((N_NODES, D_FEAT), jnp.float32),
    ] + [pltpu.SemaphoreType.DMA] * (3 * DEPTH),
)(_sc_body)


def _merge_body(p_ref, o_ref):
    o_ref[...] = p_ref[0] + p_ref[1]


MERGE_BLK = 2000

_merge = pl.pallas_call(
    _merge_body,
    grid=(N_NODES // MERGE_BLK,),
    in_specs=[pl.BlockSpec((NC, MERGE_BLK, D_FEAT), lambda i: (0, i, 0))],
    out_specs=pl.BlockSpec((MERGE_BLK, D_FEAT), lambda i: (i, 0)),
    out_shape=jax.ShapeDtypeStruct((N_NODES, D_FEAT), jnp.float32),
)


@jax.jit
def kernel(h, edge_index, edge_weight):
    wr = edge_weight.reshape(1, N_EDGES)
    partials = _sc_call(h, edge_index.astype(jnp.int32), wr)
    return _merge(partials)


# speedup vs baseline: 1.0058x; 1.0058x over previous
import dataclasses
import functools

import jax
import jax.numpy as jnp
from jax import lax
from jax.experimental import pallas as pl
from jax.experimental.pallas import tpu as pltpu
from jax.experimental.pallas import tpu_sc as plsc

N_NODES = 10000
N_EDGES = 320000
D_FEAT = 128

NC = 2
NS = 16
NW = NC * NS
CHUNK = 128
TOTAL_CHUNKS = N_EDGES // CHUNK
MAX_CHUNKS = -(-TOTAL_CHUNKS // NW)
LANES = 16
DEPTH = 3
WO_ROWS = 80


def _sc_body(h_hbm, ei_hbm, w_hbm, out_hbm, idx_v, w_v, rows_v, acc_sh,
             *sems):
    isem = sems[0:DEPTH]
    gsem = sems[DEPTH:2 * DEPTH]
    ssem = sems[2 * DEPTH:3 * DEPTH]

    c = lax.axis_index("c")
    s = lax.axis_index("s")
    wid = c * NS + s
    nchunks = jnp.where(wid < TOTAL_CHUNKS - (MAX_CHUNKS - 1) * NW,
                        MAX_CHUNKS, MAX_CHUNKS - 1)

    def issue_idx(n, j):
        off = (wid + n * NW) * CHUNK
        pltpu.async_copy(ei_hbm.at[:, pl.ds(off, CHUNK)],
                         idx_v.at[j], isem[j])
        pltpu.async_copy(w_hbm.at[0, pl.ds(off, CHUNK)], w_v.at[j], isem[j])

    def wait_idx(j):
        pltpu.make_async_copy(ei_hbm.at[:, pl.ds(0, CHUNK)], idx_v.at[j],
                              isem[j]).wait()
        pltpu.make_async_copy(w_hbm.at[0, pl.ds(0, CHUNK)], w_v.at[j],
                              isem[j]).wait()

    def issue_gather(j):
        pltpu.async_copy(h_hbm.at[idx_v.at[j, 0]], rows_v.at[j], gsem[j])

    def wait_gather(j):
        pltpu.make_async_copy(h_hbm.at[idx_v.at[j, 0]], rows_v.at[j],
                              gsem[j]).wait()

    def issue_scatter(j):
        pltpu.async_copy(rows_v.at[j], acc_sh.at[idx_v.at[j, 1]], ssem[j],
                         add=True)

    def wait_scatter(j):
        pltpu.make_async_copy(rows_v.at[j], acc_sh.at[idx_v.at[j, 1]],
                              ssem[j]).wait()

    issue_idx(0, 0)
    issue_idx(1, 1)
    wait_idx(0)
    issue_gather(0)

    row_base = s * 640
    wo_trips = jnp.where(s == NS - 1, 5, 8)

    @pl.loop(0, WO_ROWS)
    def _zero_stage(i):
        for k in range(D_FEAT // LANES):
            rows_v[2, i, pl.ds(k * LANES, LANES)] = jnp.zeros((LANES,),
                                                              jnp.float32)

    def _zero_acc(j, _):
        pltpu.sync_copy(rows_v.at[2, pl.ds(0, WO_ROWS)],
                        acc_sh.at[pl.ds(row_base + j * WO_ROWS, WO_ROWS)])
        return _

    lax.fori_loop(0, wo_trips, _zero_acc, None)

    plsc.subcore_barrier()

    @pl.loop(0, (MAX_CHUNKS + DEPTH - 1) // DEPTH)
    def _triple(m):
        for b in range(DEPTH):
            n = m * DEPTH + b

            @pl.when(n < nchunks)
            def _():
                @pl.when(n + 1 < nchunks)
                def _():
                    wait_idx((b + 1) % DEPTH)
                    issue_gather((b + 1) % DEPTH)

                wait_gather(b)

                @pl.when(n >= 1)
                def _():
                    wait_scatter((b + 2) % DEPTH)

                @pl.when(n + 2 < nchunks)
                def _():
                    issue_idx(n + 2, (b + 2) % DEPTH)

                @pl.loop(0, CHUNK // LANES)
                def _scale(g):
                    wv = w_v[b, pl.ds(g * LANES, LANES)]
                    for jj in range(LANES):
                        w = wv[jj]
                        i = g * LANES + jj
                        for k in range(D_FEAT // LANES):
                            sl = pl.ds(k * LANES, LANES)
                            rows_v[b, i, sl] = rows_v[b, i, sl] * w

                issue_scatter(b)

    @pl.when(nchunks == MAX_CHUNKS)
    def _():
        wait_scatter((MAX_CHUNKS - 1) % DEPTH)

    @pl.when(nchunks == MAX_CHUNKS - 1)
    def _():
        wait_scatter((MAX_CHUNKS - 2) % DEPTH)

    plsc.subcore_barrier()

    def _writeout(j, _):
        row0 = row_base + j * WO_ROWS
        pltpu.sync_copy(acc_sh.at[pl.ds(row0, WO_ROWS)],
                        out_hbm.at[c, pl.ds(row0, WO_ROWS)])
        return _

    lax.fori_loop(0, wo_trips, _writeout, None)


_cp = pltpu.CompilerParams()
if "needs_layout_passes" in pltpu.CompilerParams.__dataclass_fields__:
    _cp = dataclasses.replace(_cp, needs_layout_passes=False)

_sc_call = functools.partial(
    pl.kernel,
    compiler_params=_cp,
    out_type=jax.ShapeDtypeStruct((NC, N_NODES, D_FEAT), jnp.float32),
    mesh=plsc.VectorSubcoreMesh(core_axis_name="c", subcore_axis_name="s"),
    scratch_types=[
        pltpu.VMEM((DEPTH, 2, CHUNK), jnp.int32),
        pltpu.VMEM((DEPTH, CHUNK), jnp.float32),
        pltpu.VMEM((DEPTH, CHUNK, D_FEAT), jnp.float32),
        pltpu.VMEM_SHARED((N_NODES, D_FEAT), jnp.float32),
    ] + [pltpu.SemaphoreType.DMA] * (3 * DEPTH),
)(_sc_body)


def _merge_body(p_ref, o_ref):
    o_ref[...] = p_ref[0] + p_ref[1]


MERGE_BLK = 2000

_merge = pl.pallas_call(
    _merge_body,
    grid=(N_NODES // MERGE_BLK,),
    in_specs=[pl.BlockSpec((NC, MERGE_BLK, D_FEAT), lambda i: (0, i, 0))],
    out_specs=pl.BlockSpec((MERGE_BLK, D_FEAT), lambda i: (i, 0)),
    out_shape=jax.ShapeDtypeStruct((N_NODES, D_FEAT), jnp.float32),
)


@jax.jit
def kernel(h, edge_index, edge_weight):
    wr = edge_weight.reshape(1, N_EDGES)
    partials = _sc_call(h, edge_index.astype(jnp.int32), wr)
    return _merge(partials)
